# Initial kernel scaffold; baseline (speedup 1.0000x reference)
import jax
import jax.numpy as jnp
from jax.experimental import pallas as pl


def kernel(x, edge_index, donor_emb, W1, a_src1, a_dst1, b1, W2, a_src2, a_dst2, b2, S1w, S1b, S2w, S2b):
    raise NotImplementedError("write your pallas kernel here")


# speedup vs baseline: 1.0392x; 1.0392x over previous
import functools
import jax
import jax.numpy as jnp
from jax.experimental import pallas as pl
from jax.experimental.pallas import tpu as pltpu

N = 50000
E = 800000
NODE_DIM = 27
HID = 64
CTX = 32
HEADS = 4
NP = 50048


def _dense1_body(x_ref, w_ref, asrc_ref, adst_ref, h_ref, als_ref, ald_ref):
    x = x_ref[...]
    w = w_ref[...]
    h = jnp.dot(x, w, preferred_element_type=jnp.float32)
    h_ref[...] = h
    hh = h.reshape(x.shape[0], HEADS, HID)
    als_ref[...] = jnp.sum(hh * asrc_ref[...][None], axis=-1)
    ald_ref[...] = jnp.sum(hh * adst_ref[...][None], axis=-1)


def _dense_proj(x, W, a_src, a_dst):
    B = 3128
    grid = NP // B
    return pl.pallas_call(
        _dense1_body,
        grid=(grid,),
        in_specs=[
            pl.BlockSpec((B, x.shape[1]), lambda i: (i, 0)),
            pl.BlockSpec((x.shape[1], HEADS * HID), lambda i: (0, 0)),
            pl.BlockSpec((HEADS, HID), lambda i: (0, 0)),
            pl.BlockSpec((HEADS, HID), lambda i: (0, 0)),
        ],
        out_specs=[
            pl.BlockSpec((B, HEADS * HID), lambda i: (i, 0)),
            pl.BlockSpec((B, HEADS), lambda i: (i, 0)),
            pl.BlockSpec((B, HEADS), lambda i: (i, 0)),
        ],
        out_shape=[
            jax.ShapeDtypeStruct((NP, HEADS * HID), jnp.float32),
            jax.ShapeDtypeStruct((NP, HEADS), jnp.float32),
            jax.ShapeDtypeStruct((NP, HEADS), jnp.float32),
        ],
    )(x, W, a_src, a_dst)


def _finalize_body(num_ref, s_ref, b_ref, out_ref):
    num = num_ref[...]
    B = num.shape[0]
    s = s_ref[...]
    out = num.reshape(B, HEADS, HID) / (s[:, :, None] + 1e-16)
    m = jnp.mean(out, axis=1) + b_ref[...][None, :]
    out_ref[...] = jnp.where(m > 0, m, jnp.exp(jnp.minimum(m, 0.0)) - 1.0)


def _finalize(num, s, b):
    B = 3128
    grid = NP // B
    return pl.pallas_call(
        _finalize_body,
        grid=(grid,),
        in_specs=[
            pl.BlockSpec((B, HEADS * HID), lambda i: (i, 0)),
            pl.BlockSpec((B, HEADS), lambda i: (i, 0)),
            pl.BlockSpec((HID,), lambda i: (0,)),
        ],
        out_specs=pl.BlockSpec((B, HID), lambda i: (i, 0)),
        out_shape=jax.ShapeDtypeStruct((NP, HID), jnp.float32),
    )(num, s, b)


def _scorer_body(h_ref, demb_ref, s1w_ref, s1b_ref, s2w_ref, s2b_ref, out_ref):
    h = h_ref[...]
    ctx_w = s1w_ref[...]
    hid = jnp.dot(h, ctx_w[:HID], preferred_element_type=jnp.float32)
    hid += jnp.dot(demb_ref[...][None, :], ctx_w[HID:], preferred_element_type=jnp.float32)
    hid = jax.nn.relu(hid + s1b_ref[...][None, :])
    out_ref[...] = (jnp.dot(hid, s2w_ref[...], preferred_element_type=jnp.float32)
                    + s2b_ref[...][None, :])


def _scorer(h, donor_emb, S1w, S1b, S2w, S2b):
    B = 3128
    grid = NP // B
    return pl.pallas_call(
        _scorer_body,
        grid=(grid,),
        in_specs=[
            pl.BlockSpec((B, HID), lambda i: (i, 0)),
            pl.BlockSpec((CTX,), lambda i: (0,)),
            pl.BlockSpec((HID + CTX, HID), lambda i: (0, 0)),
            pl.BlockSpec((HID,), lambda i: (0,)),
            pl.BlockSpec((HID, 1), lambda i: (0, 0)),
            pl.BlockSpec((1,), lambda i: (0,)),
        ],
        out_specs=pl.BlockSpec((B, 1), lambda i: (i, 0)),
        out_shape=jax.ShapeDtypeStruct((NP, 1), jnp.float32),
    )(h, donor_emb, S1w, S1b, S2w, S2b)


def _edge_pass(h, als, ald, src, dst):
    C = jax.nn.leaky_relu(als[:N].max(0) + ald[:N].max(0), 0.2)
    e = jax.nn.leaky_relu(als[src] + ald[dst], 0.2)
    ex = jnp.exp(e - C[None, :])
    s = jax.ops.segment_sum(ex, dst, num_segments=NP)
    hh = h.reshape(NP, HEADS, HID)
    num = jax.ops.segment_sum(ex[:, :, None] * hh[src], dst, num_segments=NP)
    return num.reshape(NP, HEADS * HID), s


def kernel(x, edge_index, donor_emb, W1, a_src1, a_dst1, b1, W2, a_src2, a_dst2, b2, S1w, S1b, S2w, S2b):
    loops = jnp.arange(N, dtype=edge_index.dtype)
    src = jnp.concatenate([edge_index[0], loops])
    dst = jnp.concatenate([edge_index[1], loops])

    xp = jnp.zeros((NP, NODE_DIM), jnp.float32).at[:N].set(x)
    h1, als1, ald1 = _dense_proj(xp, W1, a_src1, a_dst1)
    num1, s1 = _edge_pass(h1, als1, ald1, src, dst)
    hm1 = _finalize(num1, s1, b1)

    h2, als2, ald2 = _dense_proj(hm1, W2, a_src2, a_dst2)
    num2, s2 = _edge_pass(h2, als2, ald2, src, dst)
    hm2 = _finalize(num2, s2, b2)

    logits = _scorer(hm2, donor_emb, S1w, S1b, S2w, S2b)
    return logits[:N, 0]


# speedup vs baseline: 1.3769x; 1.3249x over previous
import jax
import jax.numpy as jnp
from jax import lax
from jax.experimental import pallas as pl
from jax.experimental.pallas import tpu as pltpu
from jax.experimental.pallas import tpu_sc as plsc

N = 50000
NP = 50176
E = 800000
EDGES = E + N
NODE_DIM = 27
HID = 64
CTX = 32
HEADS = 4
F4 = HEADS * HID

BLK = 1024
EP = 851968
NBLK_ATT = 26
NBLK_MSG = 52
SH_MSG = BLK * NBLK_MSG
CH = 1792
NCH = 14
STR = CH // 16
BATCH = 128
NR = 10
NEG = -1e30


def _proj_body(x_ref, w_ref, asrc_ref, adst_ref,
               h_ref, als_ref, ald_ref, maxs_ref, maxd_ref):
    i = pl.program_id(0)
    B = x_ref.shape[0]
    h = jnp.dot(x_ref[...], w_ref[...], preferred_element_type=jnp.float32)
    h_ref[...] = h
    hh = h.reshape(B, HEADS, HID)
    als = jnp.sum(hh * asrc_ref[...][None], axis=-1)
    ald = jnp.sum(hh * adst_ref[...][None], axis=-1)
    row = i * B + lax.broadcasted_iota(jnp.int32, (B, 1), 0)
    valid = row < N
    als16 = jnp.concatenate([als, jnp.zeros((B, 16 - HEADS), jnp.float32)], axis=1)
    ald16 = jnp.concatenate([ald, jnp.zeros((B, 16 - HEADS), jnp.float32)], axis=1)
    als16 = jnp.where(valid, als16, NEG)
    ald16 = jnp.where(valid, ald16, NEG)
    als_ref[...] = als16
    ald_ref[...] = ald16
    ms = jnp.max(als16, axis=0)
    md = jnp.max(ald16, axis=0)

    @pl.when(i == 0)
    def _():
        maxs_ref[...] = ms
        maxd_ref[...] = md

    @pl.when(i > 0)
    def _():
        maxs_ref[...] = jnp.maximum(maxs_ref[...], ms)
        maxd_ref[...] = jnp.maximum(maxd_ref[...], md)


def _proj(xp, W, a_src, a_dst):
    B = NP // 16
    D = xp.shape[1]
    return pl.pallas_call(
        _proj_body,
        grid=(16,),
        in_specs=[
            pl.BlockSpec((B, D), lambda i: (i, 0)),
            pl.BlockSpec((D, F4), lambda i: (0, 0)),
            pl.BlockSpec((HEADS, HID), lambda i: (0, 0)),
            pl.BlockSpec((HEADS, HID), lambda i: (0, 0)),
        ],
        out_specs=[
            pl.BlockSpec((B, F4), lambda i: (i, 0)),
            pl.BlockSpec((B, 16), lambda i: (i, 0)),
            pl.BlockSpec((B, 16), lambda i: (i, 0)),
            pl.BlockSpec((16,), lambda i: (0,)),
            pl.BlockSpec((16,), lambda i: (0,)),
        ],
        out_shape=[
            jax.ShapeDtypeStruct((NP, F4), jnp.float32),
            jax.ShapeDtypeStruct((NP, 16), jnp.float32),
            jax.ShapeDtypeStruct((NP, 16), jnp.float32),
            jax.ShapeDtypeStruct((16,), jnp.float32),
            jax.ShapeDtypeStruct((16,), jnp.float32),
        ],
    )(xp, W, a_src, a_dst)


def _att_body(src_hbm, dst_hbm, als_hbm, ald_hbm, maxs_hbm, maxd_hbm,
              exs_hbm,
              srcv, dstv, abuf, bbuf, exbuf, mbuf):
    c = lax.axis_index("c")
    sid = lax.axis_index("s")
    wid = sid * 2 + c
    base = wid * (EP // 32)

    pltpu.sync_copy(maxs_hbm, mbuf)
    ms = mbuf[...]
    pltpu.sync_copy(maxd_hbm, mbuf)
    md = mbuf[...]
    w = ms + md
    cub = jnp.where(w > 0, w, 0.2 * w)

    def p1_block(b, carry):
        offr = pl.multiple_of(base // 128 + b * (BLK // 128), 8)
        pltpu.sync_copy(src_hbm.at[pl.ds(offr, BLK // 128)], srcv)
        pltpu.sync_copy(dst_hbm.at[pl.ds(offr, BLK // 128)], dstv)
        for r in range(BLK // 128):
            pltpu.sync_copy(als_hbm.at[srcv.at[r]], abuf.at[pl.ds(128 * r, 128)])
            pltpu.sync_copy(ald_hbm.at[dstv.at[r]], bbuf.at[pl.ds(128 * r, 128)])

        def edge(i, carry2):
            v = abuf[i, :] + bbuf[i, :]
            v = jnp.where(v > 0, v, 0.2 * v) - cub
            exbuf[i, :] = jnp.exp(v)
            return carry2
        lax.fori_loop(0, BLK, edge, 0)
        pltpu.sync_copy(exbuf, exs_hbm.at[pl.ds(pl.multiple_of(base + b * BLK, 8), BLK)])
        return carry
    lax.fori_loop(0, NBLK_ATT, p1_block, 0)


def _sc_att(src2d, dst2d, als16, ald16, maxs, maxd):
    mesh = plsc.VectorSubcoreMesh(core_axis_name="c", subcore_axis_name="s")
    fn = pl.kernel(
        _att_body,
        out_type=[jax.ShapeDtypeStruct((EP, 16), jnp.float32)],
        mesh=mesh,
        compiler_params=pltpu.CompilerParams(use_tc_tiling_on_sc=False),
        scratch_types=[
            pltpu.VMEM((BLK // 128, 128), jnp.int32),
            pltpu.VMEM((BLK // 128, 128), jnp.int32),
            pltpu.VMEM((BLK, 16), jnp.float32),
            pltpu.VMEM((BLK, 16), jnp.float32),
            pltpu.VMEM((BLK, 16), jnp.float32),
            pltpu.VMEM((16,), jnp.float32),
        ],
    )
    return fn(src2d, dst2d, als16, ald16, maxs, maxd)[0]


def _msg_body(src_hbm, dst_hbm, exs_hbm, h_hbm,
              num_hbm, s16_hbm,
              srcv, dstv, selsrc, seldl, seleid, hrow, exrow, zb256, zb16,
              pendbuf, acc, s16acc):
    c = lax.axis_index("c")
    sid = lax.axis_index("s")
    base = sid * SH_MSG
    lane = lax.iota(jnp.int32, 16)
    one = jnp.ones((16,), jnp.int32)
    padeid = jnp.full((16,), EP - 1, jnp.int32)
    zero16i = jnp.zeros((16,), jnp.int32)
    c15 = jnp.full((16,), 15, jnp.int32)

    def _zrow(i, carry):
        for jj in range(16):
            zb256[i, pl.ds(16 * jj, 16)] = jnp.zeros((16,), jnp.float32)
        return carry
    lax.fori_loop(0, 32, _zrow, 0)

    def _zrow16(i, carry):
        zb16[i, :] = jnp.zeros((16,), jnp.float32)
        return carry
    lax.fori_loop(0, STR, _zrow16, 0)

    for j in range(NCH):
        g = c * NCH + j
        lo = g * CH
        hi = lo + CH

        for k in range(4):
            st = pl.multiple_of(jnp.minimum(sid * STR + 32 * k, CH - 32), 8)
            pltpu.sync_copy(zb256, acc.at[pl.ds(st, 32)])
        pltpu.sync_copy(zb16, s16acc.at[pl.ds(pl.multiple_of(sid * STR, 8), STR)])
        plsc.subcore_barrier()

        def p2_block(b, carry):
            off = base + b * BLK
            offr = pl.multiple_of(off // 128, 8)
            pltpu.sync_copy(src_hbm.at[pl.ds(offr, BLK // 128)], srcv)
            pltpu.sync_copy(dst_hbm.at[pl.ds(offr, BLK // 128)], dstv)

            def grp(i, st2):
                nrows, pcnt = st2
                pcntv = jnp.full((16,), pcnt, jnp.int32)
                psrc = pendbuf[0, :]
                pdl = pendbuf[1, :]
                peid = pendbuf[2, :]
                r = i // 8
                col = (i % 8) * 16
                dstw = dstv[r, pl.ds(col, 16)]
                srcw = srcv[r, pl.ds(col, 16)]
                gelo = jnp.minimum(jnp.maximum(dstw - lo + 1, 0), 1)
                lthi = jnp.minimum(jnp.maximum(hi - dstw, 0), 1)
                mi = gelo * lthi
                pf = mi
                for sh in (1, 2, 4, 8):
                    smask = jnp.minimum(jnp.maximum(lane - (sh - 1), 0), 1)
                    pf = pf + pf.at[jnp.maximum(lane - sh, 0)].get(mode="promise_in_bounds") * smask
                pos = jnp.full((16,), -1, jnp.int32)
                for step in (8, 4, 2, 1):
                    cand = pos + step
                    pfc = pf.at[cand].get(mode="promise_in_bounds")
                    adv = jnp.minimum(jnp.maximum(lane - pfc + 1, 0), 1)
                    pos = pos + step * adv
                perm = jnp.minimum(pos + 1, 15)
                eidw = jnp.full((16,), off + 16 * i, jnp.int32) + lane
                csrc = srcw.at[perm].get(mode="promise_in_bounds")
                cdl = (dstw - lo).at[perm].get(mode="promise_in_bounds")
                ceid = eidw.at[perm].get(mode="promise_in_bounds")
                sh_idx = jnp.maximum(lane - pcntv, 0)
                ge = jnp.minimum(jnp.maximum(lane - pcntv + 1, 0), 1)
                mg_src = csrc.at[sh_idx].get(mode="promise_in_bounds") * ge + psrc * (one - ge)
                mg_dl = cdl.at[sh_idx].get(mode="promise_in_bounds") * ge + pdl * (one - ge)
                mg_eid = ceid.at[sh_idx].get(mode="promise_in_bounds") * ge + peid * (one - ge)
                r2 = nrows // 8
                c2 = 16 * (nrows % 8)
                selsrc[r2, pl.ds(c2, 16)] = mg_src
                seldl[r2, pl.ds(c2, 16)] = mg_dl
                seleid[r2, pl.ds(c2, 16)] = mg_eid
                cnt = pf[15]
                newt_s = pcnt + cnt
                fi_s = jnp.minimum(jnp.maximum(newt_s - 15, 0), 1)
                fi = jnp.full((16,), fi_s, jnp.int32)
                lf_idx = jnp.minimum(lane + 16 - pcntv, 15)
                lf_src = csrc.at[lf_idx].get(mode="promise_in_bounds")
                lf_dl = cdl.at[lf_idx].get(mode="promise_in_bounds")
                lf_eid = ceid.at[lf_idx].get(mode="promise_in_bounds")
                pendbuf[0, :] = lf_src * fi + mg_src * (one - fi)
                pendbuf[1, :] = lf_dl * fi + mg_dl * (one - fi)
                pendbuf[2, :] = lf_eid * fi + mg_eid * (one - fi)
                return (nrows + fi_s, newt_s - 16 * fi_s)

            pendbuf[0, :] = zero16i
            pendbuf[1, :] = zero16i
            pendbuf[2, :] = padeid
            nrows, pcnt = lax.fori_loop(0, BLK // 16, grp,
                                        (jnp.int32(0), jnp.int32(0)))

            pcntv = jnp.full((16,), pcnt, jnp.int32)
            vm = jnp.minimum(jnp.maximum(pcntv - lane, 0), 1)
            r2 = nrows // 8
            c2 = 16 * (nrows % 8)
            selsrc[r2, pl.ds(c2, 16)] = pendbuf[0, :] * vm
            seldl[r2, pl.ds(c2, 16)] = pendbuf[1, :] * vm
            seleid[r2, pl.ds(c2, 16)] = pendbuf[2, :] * vm + padeid * (one - vm)
            for kk in range(1, 8):
                idxp = jnp.minimum(nrows + kk, NR * 8 - 1)
                rp = idxp // 8
                cp = 16 * (idxp % 8)
                selsrc[rp, pl.ds(cp, 16)] = zero16i
                seldl[rp, pl.ds(cp, 16)] = zero16i
                seleid[rp, pl.ds(cp, 16)] = padeid
            total = nrows * 16 + pcnt
            nb = (total + BATCH - 1) // BATCH

            def batch(k, carry2):
                pltpu.sync_copy(h_hbm.at[selsrc.at[k]], hrow)
                pltpu.sync_copy(exs_hbm.at[seleid.at[k]], exrow)

                def sc_edge(i, carry3):
                    ev = exrow[i, :]
                    for hh in range(HEADS):
                        bv = jnp.full((16,), ev[hh], jnp.float32)
                        for vv in range(4):
                            sl = pl.ds(hh * HID + vv * 16, 16)
                            hrow[i, sl] = hrow[i, sl] * bv
                    return carry3
                lax.fori_loop(0, BATCH, sc_edge, 0)
                pltpu.sync_copy(hrow, acc.at[seldl.at[k]], add=True)
                pltpu.sync_copy(exrow, s16acc.at[seldl.at[k]], add=True)
                return carry2

            lax.fori_loop(0, nb, batch, 0)
            return carry
        lax.fori_loop(0, NBLK_MSG, p2_block, 0)
        plsc.subcore_barrier()

        stw = pl.multiple_of(sid * STR, 8)
        dtw = pl.multiple_of(lo + sid * STR, 8)
        pltpu.sync_copy(acc.at[pl.ds(stw, STR)], num_hbm.at[pl.ds(dtw, STR)])
        pltpu.sync_copy(s16acc.at[pl.ds(stw, STR)], s16_hbm.at[pl.ds(dtw, STR)])
        plsc.subcore_barrier()


def _sc_msg(src2d, dst2d, exs, h):
    mesh = plsc.VectorSubcoreMesh(core_axis_name="c", subcore_axis_name="s")
    fn = pl.kernel(
        _msg_body,
        out_type=[
            jax.ShapeDtypeStruct((NP, F4), jnp.float32),
            jax.ShapeDtypeStruct((NP, 16), jnp.float32),
        ],
        mesh=mesh,
        compiler_params=pltpu.CompilerParams(use_tc_tiling_on_sc=False),
        scratch_types=[
            pltpu.VMEM((BLK // 128, 128), jnp.int32),
            pltpu.VMEM((BLK // 128, 128), jnp.int32),
            pltpu.VMEM((NR, 128), jnp.int32),
            pltpu.VMEM((NR, 128), jnp.int32),
            pltpu.VMEM((NR, 128), jnp.int32),
            pltpu.VMEM((BATCH, F4), jnp.float32),
            pltpu.VMEM((BATCH, 16), jnp.float32),
            pltpu.VMEM((32, F4), jnp.float32),
            pltpu.VMEM((STR, 16), jnp.float32),
            pltpu.VMEM((4, 16), jnp.int32),
            pltpu.VMEM_SHARED((CH, F4), jnp.float32),
            pltpu.VMEM_SHARED((CH, 16), jnp.float32),
        ],
    )
    return fn(src2d, dst2d, exs, h)


def _finalize_body(num_ref, s16_ref, b_ref, out_ref):
    B = num_ref.shape[0]
    s = s16_ref[...][:, :HEADS]
    out = num_ref[...].reshape(B, HEADS, HID) / (s[:, :, None] + 1e-16)
    m = jnp.mean(out, axis=1) + b_ref[...][None, :]
    out_ref[...] = jnp.where(m > 0, m, jnp.exp(jnp.minimum(m, 0.0)) - 1.0)


def _finalize(num, s16, b):
    B = NP // 16
    return pl.pallas_call(
        _finalize_body,
        grid=(16,),
        in_specs=[
            pl.BlockSpec((B, F4), lambda i: (i, 0)),
            pl.BlockSpec((B, 16), lambda i: (i, 0)),
            pl.BlockSpec((HID,), lambda i: (0,)),
        ],
        out_specs=pl.BlockSpec((B, HID), lambda i: (i, 0)),
        out_shape=jax.ShapeDtypeStruct((NP, HID), jnp.float32),
    )(num, s16, b)


def _scorer_body(h_ref, demb_ref, s1w_ref, s1b_ref, s2w_ref, s2b_ref, out_ref):
    h = h_ref[...]
    ctx_w = s1w_ref[...]
    hid = jnp.dot(h, ctx_w[:HID], preferred_element_type=jnp.float32)
    hid += jnp.dot(demb_ref[...][None, :], ctx_w[HID:], preferred_element_type=jnp.float32)
    hid = jax.nn.relu(hid + s1b_ref[...][None, :])
    out_ref[...] = (jnp.dot(hid, s2w_ref[...], preferred_element_type=jnp.float32)
                    + s2b_ref[...][None, :])


def _scorer(h, donor_emb, S1w, S1b, S2w, S2b):
    B = NP // 16
    return pl.pallas_call(
        _scorer_body,
        grid=(16,),
        in_specs=[
            pl.BlockSpec((B, HID), lambda i: (i, 0)),
            pl.BlockSpec((CTX,), lambda i: (0,)),
            pl.BlockSpec((HID + CTX, HID), lambda i: (0, 0)),
            pl.BlockSpec((HID,), lambda i: (0,)),
            pl.BlockSpec((HID, 1), lambda i: (0, 0)),
            pl.BlockSpec((1,), lambda i: (0,)),
        ],
        out_specs=pl.BlockSpec((B, 1), lambda i: (i, 0)),
        out_shape=jax.ShapeDtypeStruct((NP, 1), jnp.float32),
    )(h, donor_emb, S1w, S1b, S2w, S2b)


def _gat_layer_sc(src2d, dst2d, xp, W, a_src, a_dst, b):
    h, als16, ald16, ms, md = _proj(xp, W, a_src, a_dst)
    exs = _sc_att(src2d, dst2d, als16, ald16, ms, md)
    num, s16 = _sc_msg(src2d, dst2d, exs, h)
    return _finalize(num, s16, b)


def kernel(x, edge_index, donor_emb, W1, a_src1, a_dst1, b1, W2, a_src2, a_dst2, b2, S1w, S1b, S2w, S2b):
    loops = jnp.arange(N, dtype=jnp.int32)
    pad = jnp.full((EP - EDGES,), N, jnp.int32)
    src2d = jnp.concatenate([edge_index[0].astype(jnp.int32), loops, pad]).reshape(EP // 128, 128)
    dst2d = jnp.concatenate([edge_index[1].astype(jnp.int32), loops, pad]).reshape(EP // 128, 128)

    xp = jnp.zeros((NP, NODE_DIM), jnp.float32).at[:N].set(x)
    hm1 = _gat_layer_sc(src2d, dst2d, xp, W1, a_src1, a_dst1, b1)
    hm2 = _gat_layer_sc(src2d, dst2d, hm1, W2, a_src2, a_dst2, b2)
    logits = _scorer(hm2, donor_emb, S1w, S1b, S2w, S2b)
    return logits[:N, 0]


# speedup vs baseline: 4.3985x; 3.1945x over previous
import jax
import jax.numpy as jnp
from jax import lax
from jax.experimental import pallas as pl
from jax.experimental.pallas import tpu as pltpu
from jax.experimental.pallas import tpu_sc as plsc

N = 50000
NP = 50176
E = 800000
EDGES = E + N
NODE_DIM = 27
HID = 64
CTX = 32
HEADS = 4
F4 = HEADS * HID

BLK = 1024
EP = 851968
NBLK_ATT = 26
NBLK_MSG = 52
SH_MSG = BLK * NBLK_MSG
CH = 3584
NCH = 7
STR = CH // 16
BATCH = 128
NR = 10
NEG = -1e30


def _proj_body(x_ref, w_ref, asrc_ref, adst_ref,
               h_ref, als_ref, ald_ref, maxs_ref, maxd_ref):
    i = pl.program_id(0)
    B = x_ref.shape[0]
    h = jnp.dot(x_ref[...], w_ref[...], preferred_element_type=jnp.float32)
    h_ref[...] = h
    hh = h.reshape(B, HEADS, HID)
    als = jnp.sum(hh * asrc_ref[...][None], axis=-1)
    ald = jnp.sum(hh * adst_ref[...][None], axis=-1)
    row = i * B + lax.broadcasted_iota(jnp.int32, (B, 1), 0)
    valid = row < N
    als16 = jnp.concatenate([als, jnp.zeros((B, 16 - HEADS), jnp.float32)], axis=1)
    ald16 = jnp.concatenate([ald, jnp.zeros((B, 16 - HEADS), jnp.float32)], axis=1)
    als16 = jnp.where(valid, als16, NEG)
    ald16 = jnp.where(valid, ald16, NEG)
    als_ref[...] = als16
    ald_ref[...] = ald16
    ms = jnp.max(als16, axis=0)
    md = jnp.max(ald16, axis=0)

    @pl.when(i == 0)
    def _():
        maxs_ref[...] = ms
        maxd_ref[...] = md

    @pl.when(i > 0)
    def _():
        maxs_ref[...] = jnp.maximum(maxs_ref[...], ms)
        maxd_ref[...] = jnp.maximum(maxd_ref[...], md)


def _proj(xp, W, a_src, a_dst):
    B = NP // 16
    D = xp.shape[1]
    return pl.pallas_call(
        _proj_body,
        grid=(16,),
        in_specs=[
            pl.BlockSpec((B, D), lambda i: (i, 0)),
            pl.BlockSpec((D, F4), lambda i: (0, 0)),
            pl.BlockSpec((HEADS, HID), lambda i: (0, 0)),
            pl.BlockSpec((HEADS, HID), lambda i: (0, 0)),
        ],
        out_specs=[
            pl.BlockSpec((B, F4), lambda i: (i, 0)),
            pl.BlockSpec((B, 16), lambda i: (i, 0)),
            pl.BlockSpec((B, 16), lambda i: (i, 0)),
            pl.BlockSpec((16,), lambda i: (0,)),
            pl.BlockSpec((16,), lambda i: (0,)),
        ],
        out_shape=[
            jax.ShapeDtypeStruct((NP, F4), jnp.float32),
            jax.ShapeDtypeStruct((NP, 16), jnp.float32),
            jax.ShapeDtypeStruct((NP, 16), jnp.float32),
            jax.ShapeDtypeStruct((16,), jnp.float32),
            jax.ShapeDtypeStruct((16,), jnp.float32),
        ],
    )(xp, W, a_src, a_dst)


def _att_body(src_hbm, dst_hbm, als_hbm, ald_hbm, maxs_hbm, maxd_hbm,
              exs_hbm,
              srcv, dstv, abuf, bbuf, exbuf, mbuf):
    c = lax.axis_index("c")
    sid = lax.axis_index("s")
    wid = sid * 2 + c
    base = wid * (EP // 32)

    pltpu.sync_copy(maxs_hbm, mbuf)
    ms = mbuf[...]
    pltpu.sync_copy(maxd_hbm, mbuf)
    md = mbuf[...]
    w = ms + md
    cub = jnp.where(w > 0, w, 0.2 * w)

    def p1_block(b, carry):
        offr = pl.multiple_of(base // 128 + b * (BLK // 128), 8)
        pltpu.sync_copy(src_hbm.at[pl.ds(offr, BLK // 128)], srcv)
        pltpu.sync_copy(dst_hbm.at[pl.ds(offr, BLK // 128)], dstv)
        for r in range(BLK // 128):
            pltpu.sync_copy(als_hbm.at[srcv.at[r]], abuf.at[pl.ds(128 * r, 128)])
            pltpu.sync_copy(ald_hbm.at[dstv.at[r]], bbuf.at[pl.ds(128 * r, 128)])

        def edge(i, carry2):
            v = abuf[i, :] + bbuf[i, :]
            v = jnp.where(v > 0, v, 0.2 * v) - cub
            exbuf[i, :] = jnp.exp(v)
            return carry2
        lax.fori_loop(0, BLK, edge, 0)
        pltpu.sync_copy(exbuf, exs_hbm.at[pl.ds(pl.multiple_of(base + b * BLK, 8), BLK)])
        return carry
    lax.fori_loop(0, NBLK_ATT, p1_block, 0)


def _sc_att(src2d, dst2d, als16, ald16, maxs, maxd):
    mesh = plsc.VectorSubcoreMesh(core_axis_name="c", subcore_axis_name="s")
    fn = pl.kernel(
        _att_body,
        out_type=[jax.ShapeDtypeStruct((EP, 16), jnp.float32)],
        mesh=mesh,
        compiler_params=pltpu.CompilerParams(use_tc_tiling_on_sc=False),
        scratch_types=[
            pltpu.VMEM((BLK // 128, 128), jnp.int32),
            pltpu.VMEM((BLK // 128, 128), jnp.int32),
            pltpu.VMEM((BLK, 16), jnp.float32),
            pltpu.VMEM((BLK, 16), jnp.float32),
            pltpu.VMEM((BLK, 16), jnp.float32),
            pltpu.VMEM((16,), jnp.float32),
        ],
    )
    return fn(src2d, dst2d, als16, ald16, maxs, maxd)[0]


def _msg_body(src_hbm, dst_hbm, exs_hbm, h_hbm,
              num_hbm, s16_hbm,
              srcv, dstv, selsrc, seldl, seleid, hrow, exrow, zb256, zb16,
              pendbuf, acc, s16acc):
    c = lax.axis_index("c")
    sid = lax.axis_index("s")
    base = sid * SH_MSG
    lane = lax.iota(jnp.int32, 16)
    one = jnp.ones((16,), jnp.int32)
    padeid = jnp.full((16,), EP - 1, jnp.int32)
    zero16i = jnp.zeros((16,), jnp.int32)
    c15 = jnp.full((16,), 15, jnp.int32)

    def _zrow(i, carry):
        for jj in range(16):
            zb256[i, pl.ds(16 * jj, 16)] = jnp.zeros((16,), jnp.float32)
        return carry
    lax.fori_loop(0, 32, _zrow, 0)

    def _zrow16(i, carry):
        zb16[i, :] = jnp.zeros((16,), jnp.float32)
        return carry
    lax.fori_loop(0, STR, _zrow16, 0)

    for j in range(NCH):
        g = c * NCH + j
        lo = g * CH
        hi = lo + CH

        for k in range((STR + 31) // 32):
            st = pl.multiple_of(jnp.minimum(sid * STR + 32 * k, CH - 32), 8)
            pltpu.sync_copy(zb256, acc.at[pl.ds(st, 32)])
        pltpu.sync_copy(zb16, s16acc.at[pl.ds(pl.multiple_of(sid * STR, 8), STR)])
        plsc.subcore_barrier()

        def p2_block(b, carry):
            off = base + b * BLK
            offr = pl.multiple_of(off // 128, 8)
            pltpu.sync_copy(src_hbm.at[pl.ds(offr, BLK // 128)], srcv)
            pltpu.sync_copy(dst_hbm.at[pl.ds(offr, BLK // 128)], dstv)

            def grp(i, st2):
                nrows, pcnt = st2
                pcntv = jnp.full((16,), pcnt, jnp.int32)
                psrc = pendbuf[0, :]
                pdl = pendbuf[1, :]
                peid = pendbuf[2, :]
                r = i // 8
                col = (i % 8) * 16
                dstw = dstv[r, pl.ds(col, 16)]
                srcw = srcv[r, pl.ds(col, 16)]
                gelo = jnp.minimum(jnp.maximum(dstw - lo + 1, 0), 1)
                lthi = jnp.minimum(jnp.maximum(hi - dstw, 0), 1)
                mi = gelo * lthi
                pf = mi
                for sh in (1, 2, 4, 8):
                    smask = jnp.minimum(jnp.maximum(lane - (sh - 1), 0), 1)
                    pf = pf + pf.at[jnp.maximum(lane - sh, 0)].get(mode="promise_in_bounds") * smask
                pos = jnp.full((16,), -1, jnp.int32)
                for step in (8, 4, 2, 1):
                    cand = pos + step
                    pfc = pf.at[cand].get(mode="promise_in_bounds")
                    adv = jnp.minimum(jnp.maximum(lane - pfc + 1, 0), 1)
                    pos = pos + step * adv
                perm = jnp.minimum(pos + 1, 15)
                eidw = jnp.full((16,), off + 16 * i, jnp.int32) + lane
                csrc = srcw.at[perm].get(mode="promise_in_bounds")
                cdl = (dstw - lo).at[perm].get(mode="promise_in_bounds")
                ceid = eidw.at[perm].get(mode="promise_in_bounds")
                sh_idx = jnp.maximum(lane - pcntv, 0)
                ge = jnp.minimum(jnp.maximum(lane - pcntv + 1, 0), 1)
                mg_src = csrc.at[sh_idx].get(mode="promise_in_bounds") * ge + psrc * (one - ge)
                mg_dl = cdl.at[sh_idx].get(mode="promise_in_bounds") * ge + pdl * (one - ge)
                mg_eid = ceid.at[sh_idx].get(mode="promise_in_bounds") * ge + peid * (one - ge)
                r2 = nrows // 8
                c2 = 16 * (nrows % 8)
                selsrc[r2, pl.ds(c2, 16)] = mg_src
                seldl[r2, pl.ds(c2, 16)] = mg_dl
                seleid[r2, pl.ds(c2, 16)] = mg_eid
                cnt = pf[15]
                newt_s = pcnt + cnt
                fi_s = jnp.minimum(jnp.maximum(newt_s - 15, 0), 1)
                fi = jnp.full((16,), fi_s, jnp.int32)
                lf_idx = jnp.minimum(lane + 16 - pcntv, 15)
                lf_src = csrc.at[lf_idx].get(mode="promise_in_bounds")
                lf_dl = cdl.at[lf_idx].get(mode="promise_in_bounds")
                lf_eid = ceid.at[lf_idx].get(mode="promise_in_bounds")
                pendbuf[0, :] = lf_src * fi + mg_src * (one - fi)
                pendbuf[1, :] = lf_dl * fi + mg_dl * (one - fi)
                pendbuf[2, :] = lf_eid * fi + mg_eid * (one - fi)
                return (nrows + fi_s, newt_s - 16 * fi_s)

            pendbuf[0, :] = zero16i
            pendbuf[1, :] = zero16i
            pendbuf[2, :] = padeid
            nrows, pcnt = lax.fori_loop(0, BLK // 16, grp,
                                        (jnp.int32(0), jnp.int32(0)))

            pcntv = jnp.full((16,), pcnt, jnp.int32)
            vm = jnp.minimum(jnp.maximum(pcntv - lane, 0), 1)
            r2 = nrows // 8
            c2 = 16 * (nrows % 8)
            selsrc[r2, pl.ds(c2, 16)] = pendbuf[0, :] * vm
            seldl[r2, pl.ds(c2, 16)] = pendbuf[1, :] * vm
            seleid[r2, pl.ds(c2, 16)] = pendbuf[2, :] * vm + padeid * (one - vm)
            for kk in range(1, 8):
                idxp = jnp.minimum(nrows + kk, NR * 8 - 1)
                rp = idxp // 8
                cp = 16 * (idxp % 8)
                selsrc[rp, pl.ds(cp, 16)] = zero16i
                seldl[rp, pl.ds(cp, 16)] = zero16i
                seleid[rp, pl.ds(cp, 16)] = padeid
            total = nrows * 16 + pcnt
            nb = (total + BATCH - 1) // BATCH

            def batch(k, carry2):
                pltpu.sync_copy(h_hbm.at[selsrc.at[k]], hrow)
                pltpu.sync_copy(exs_hbm.at[seleid.at[k]], exrow)

                def sc_edge(i, carry3):
                    ev = exrow[i, :]
                    for hh in range(HEADS):
                        bv = jnp.full((16,), ev[hh], jnp.float32)
                        for vv in range(4):
                            sl = pl.ds(hh * HID + vv * 16, 16)
                            hrow[i, sl] = hrow[i, sl] * bv
                    return carry3
                lax.fori_loop(0, BATCH, sc_edge, 0)
                pltpu.sync_copy(hrow, acc.at[seldl.at[k]], add=True)
                pltpu.sync_copy(exrow, s16acc.at[seldl.at[k]], add=True)
                return carry2

            lax.fori_loop(0, nb, batch, 0)
            return carry
        lax.fori_loop(0, NBLK_MSG, p2_block, 0)
        plsc.subcore_barrier()

        stw = pl.multiple_of(sid * STR, 8)
        dtw = pl.multiple_of(lo + sid * STR, 8)
        pltpu.sync_copy(acc.at[pl.ds(stw, STR)], num_hbm.at[pl.ds(dtw, STR)])
        pltpu.sync_copy(s16acc.at[pl.ds(stw, STR)], s16_hbm.at[pl.ds(dtw, STR)])
        plsc.subcore_barrier()


def _sc_msg(src2d, dst2d, exs, h):
    mesh = plsc.VectorSubcoreMesh(core_axis_name="c", subcore_axis_name="s")
    fn = pl.kernel(
        _msg_body,
        out_type=[
            jax.ShapeDtypeStruct((NP, F4), jnp.float32),
            jax.ShapeDtypeStruct((NP, 16), jnp.float32),
        ],
        mesh=mesh,
        compiler_params=pltpu.CompilerParams(use_tc_tiling_on_sc=False),
        scratch_types=[
            pltpu.VMEM((BLK // 128, 128), jnp.int32),
            pltpu.VMEM((BLK // 128, 128), jnp.int32),
            pltpu.VMEM((NR, 128), jnp.int32),
            pltpu.VMEM((NR, 128), jnp.int32),
            pltpu.VMEM((NR, 128), jnp.int32),
            pltpu.VMEM((BATCH, F4), jnp.float32),
            pltpu.VMEM((BATCH, 16), jnp.float32),
            pltpu.VMEM((32, F4), jnp.float32),
            pltpu.VMEM((STR, 16), jnp.float32),
            pltpu.VMEM((4, 16), jnp.int32),
            pltpu.VMEM_SHARED((CH, F4), jnp.float32),
            pltpu.VMEM_SHARED((CH, 16), jnp.float32),
        ],
    )
    return fn(src2d, dst2d, exs, h)


def _finalize_body(num_ref, s16_ref, b_ref, out_ref):
    B = num_ref.shape[0]
    s = s16_ref[...][:, :HEADS]
    out = num_ref[...].reshape(B, HEADS, HID) / (s[:, :, None] + 1e-16)
    m = jnp.mean(out, axis=1) + b_ref[...][None, :]
    out_ref[...] = jnp.where(m > 0, m, jnp.exp(jnp.minimum(m, 0.0)) - 1.0)


def _finalize(num, s16, b):
    B = NP // 16
    return pl.pallas_call(
        _finalize_body,
        grid=(16,),
        in_specs=[
            pl.BlockSpec((B, F4), lambda i: (i, 0)),
            pl.BlockSpec((B, 16), lambda i: (i, 0)),
            pl.BlockSpec((HID,), lambda i: (0,)),
        ],
        out_specs=pl.BlockSpec((B, HID), lambda i: (i, 0)),
        out_shape=jax.ShapeDtypeStruct((NP, HID), jnp.float32),
    )(num, s16, b)


def _scorer_body(h_ref, demb_ref, s1w_ref, s1b_ref, s2w_ref, s2b_ref, out_ref):
    h = h_ref[...]
    ctx_w = s1w_ref[...]
    hid = jnp.dot(h, ctx_w[:HID], preferred_element_type=jnp.float32)
    hid += jnp.dot(demb_ref[...][None, :], ctx_w[HID:], preferred_element_type=jnp.float32)
    hid = jax.nn.relu(hid + s1b_ref[...][None, :])
    out_ref[...] = (jnp.dot(hid, s2w_ref[...], preferred_element_type=jnp.float32)
                    + s2b_ref[...][None, :])


def _scorer(h, donor_emb, S1w, S1b, S2w, S2b):
    B = NP // 16
    return pl.pallas_call(
        _scorer_body,
        grid=(16,),
        in_specs=[
            pl.BlockSpec((B, HID), lambda i: (i, 0)),
            pl.BlockSpec((CTX,), lambda i: (0,)),
            pl.BlockSpec((HID + CTX, HID), lambda i: (0, 0)),
            pl.BlockSpec((HID,), lambda i: (0,)),
            pl.BlockSpec((HID, 1), lambda i: (0, 0)),
            pl.BlockSpec((1,), lambda i: (0,)),
        ],
        out_specs=pl.BlockSpec((B, 1), lambda i: (i, 0)),
        out_shape=jax.ShapeDtypeStruct((NP, 1), jnp.float32),
    )(h, donor_emb, S1w, S1b, S2w, S2b)


def _gat_layer_sc(src2d, dst2d, xp, W, a_src, a_dst, b):
    h, als16, ald16, ms, md = _proj(xp, W, a_src, a_dst)
    exs = _sc_att(src2d, dst2d, als16, ald16, ms, md)
    num, s16 = _sc_msg(src2d, dst2d, exs, h)
    return _finalize(num, s16, b)


def kernel(x, edge_index, donor_emb, W1, a_src1, a_dst1, b1, W2, a_src2, a_dst2, b2, S1w, S1b, S2w, S2b):
    loops = jnp.arange(N, dtype=jnp.int32)
    pad = jnp.full((EP - EDGES,), N, jnp.int32)
    src2d = jnp.concatenate([edge_index[0].astype(jnp.int32), loops, pad]).reshape(EP // 128, 128)
    dst2d = jnp.concatenate([edge_index[1].astype(jnp.int32), loops, pad]).reshape(EP // 128, 128)

    xp = jnp.zeros((NP, NODE_DIM), jnp.float32).at[:N].set(x)
    hm1 = _gat_layer_sc(src2d, dst2d, xp, W1, a_src1, a_dst1, b1)
    hm2 = _gat_layer_sc(src2d, dst2d, hm1, W2, a_src2, a_dst2, b2)
    logits = _scorer(hm2, donor_emb, S1w, S1b, S2w, S2b)
    return logits[:N, 0]


# speedup vs baseline: 4.7241x; 1.0740x over previous
import jax
import jax.numpy as jnp
from jax import lax
from jax.experimental import pallas as pl
from jax.experimental.pallas import tpu as pltpu
from jax.experimental.pallas import tpu_sc as plsc

N = 50000
NP = 50176
E = 800000
EDGES = E + N
NODE_DIM = 27
HID = 64
CTX = 32
HEADS = 4
F4 = HEADS * HID

BLK = 2048
EP = 851968
NBLK_ATT = 13
NBLK_MSG = 26
SH_MSG = BLK * NBLK_MSG
CH = 3584
NCH = 7
STR = CH // 16
BATCH = 128
NR = 18
NEG = -1e30


def _proj_body(x_ref, w_ref, asrc_ref, adst_ref,
               h_ref, als_ref, ald_ref, maxs_ref, maxd_ref):
    i = pl.program_id(0)
    B = x_ref.shape[0]
    h = jnp.dot(x_ref[...], w_ref[...], preferred_element_type=jnp.float32)
    h_ref[...] = h
    hh = h.reshape(B, HEADS, HID)
    als = jnp.sum(hh * asrc_ref[...][None], axis=-1)
    ald = jnp.sum(hh * adst_ref[...][None], axis=-1)
    row = i * B + lax.broadcasted_iota(jnp.int32, (B, 1), 0)
    valid = row < N
    als16 = jnp.concatenate([als, jnp.zeros((B, 16 - HEADS), jnp.float32)], axis=1)
    ald16 = jnp.concatenate([ald, jnp.zeros((B, 16 - HEADS), jnp.float32)], axis=1)
    als16 = jnp.where(valid, als16, NEG)
    ald16 = jnp.where(valid, ald16, NEG)
    als_ref[...] = als16
    ald_ref[...] = ald16
    ms = jnp.max(als16, axis=0)
    md = jnp.max(ald16, axis=0)

    @pl.when(i == 0)
    def _():
        maxs_ref[...] = ms
        maxd_ref[...] = md

    @pl.when(i > 0)
    def _():
        maxs_ref[...] = jnp.maximum(maxs_ref[...], ms)
        maxd_ref[...] = jnp.maximum(maxd_ref[...], md)


def _proj(xp, W, a_src, a_dst):
    B = NP // 16
    D = xp.shape[1]
    return pl.pallas_call(
        _proj_body,
        grid=(16,),
        in_specs=[
            pl.BlockSpec((B, D), lambda i: (i, 0)),
            pl.BlockSpec((D, F4), lambda i: (0, 0)),
            pl.BlockSpec((HEADS, HID), lambda i: (0, 0)),
            pl.BlockSpec((HEADS, HID), lambda i: (0, 0)),
        ],
        out_specs=[
            pl.BlockSpec((B, F4), lambda i: (i, 0)),
            pl.BlockSpec((B, 16), lambda i: (i, 0)),
            pl.BlockSpec((B, 16), lambda i: (i, 0)),
            pl.BlockSpec((16,), lambda i: (0,)),
            pl.BlockSpec((16,), lambda i: (0,)),
        ],
        out_shape=[
            jax.ShapeDtypeStruct((NP, F4), jnp.float32),
            jax.ShapeDtypeStruct((NP, 16), jnp.float32),
            jax.ShapeDtypeStruct((NP, 16), jnp.float32),
            jax.ShapeDtypeStruct((16,), jnp.float32),
            jax.ShapeDtypeStruct((16,), jnp.float32),
        ],
    )(xp, W, a_src, a_dst)


def _att_body(src_hbm, dst_hbm, als_hbm, ald_hbm, maxs_hbm, maxd_hbm,
              exs_hbm,
              srcv, dstv, abuf, bbuf, exbuf, mbuf):
    c = lax.axis_index("c")
    sid = lax.axis_index("s")
    wid = sid * 2 + c
    base = wid * (EP // 32)

    pltpu.sync_copy(maxs_hbm, mbuf)
    ms = mbuf[...]
    pltpu.sync_copy(maxd_hbm, mbuf)
    md = mbuf[...]
    w = ms + md
    cub = jnp.where(w > 0, w, 0.2 * w)

    def p1_block(b, carry):
        offr = pl.multiple_of(base // 128 + b * (BLK // 128), 8)
        pltpu.sync_copy(src_hbm.at[pl.ds(offr, BLK // 128)], srcv)
        pltpu.sync_copy(dst_hbm.at[pl.ds(offr, BLK // 128)], dstv)
        for r in range(BLK // 128):
            pltpu.sync_copy(als_hbm.at[srcv.at[r]], abuf.at[pl.ds(128 * r, 128)])
            pltpu.sync_copy(ald_hbm.at[dstv.at[r]], bbuf.at[pl.ds(128 * r, 128)])

        def edge(i, carry2):
            v = abuf[i, :] + bbuf[i, :]
            v = jnp.where(v > 0, v, 0.2 * v) - cub
            exbuf[i, :] = jnp.exp(v)
            return carry2
        lax.fori_loop(0, BLK, edge, 0)
        pltpu.sync_copy(exbuf, exs_hbm.at[pl.ds(pl.multiple_of(base + b * BLK, 8), BLK)])
        return carry
    lax.fori_loop(0, NBLK_ATT, p1_block, 0)


def _sc_att(src2d, dst2d, als16, ald16, maxs, maxd):
    mesh = plsc.VectorSubcoreMesh(core_axis_name="c", subcore_axis_name="s")
    fn = pl.kernel(
        _att_body,
        out_type=[jax.ShapeDtypeStruct((EP, 16), jnp.float32)],
        mesh=mesh,
        compiler_params=pltpu.CompilerParams(use_tc_tiling_on_sc=False),
        scratch_types=[
            pltpu.VMEM((BLK // 128, 128), jnp.int32),
            pltpu.VMEM((BLK // 128, 128), jnp.int32),
            pltpu.VMEM((BLK, 16), jnp.float32),
            pltpu.VMEM((BLK, 16), jnp.float32),
            pltpu.VMEM((BLK, 16), jnp.float32),
            pltpu.VMEM((16,), jnp.float32),
        ],
    )
    return fn(src2d, dst2d, als16, ald16, maxs, maxd)[0]


def _msg_body(src_hbm, dst_hbm, exs_hbm, h_hbm,
              num_hbm, s16_hbm,
              srcv, dstv, selsrc, seldl, seleid, hrow, exrow, zb256, zb16,
              pendbuf, acc, s16acc):
    c = lax.axis_index("c")
    sid = lax.axis_index("s")
    base = sid * SH_MSG
    lane = lax.iota(jnp.int32, 16)
    one = jnp.ones((16,), jnp.int32)
    padeid = jnp.full((16,), EP - 1, jnp.int32)
    zero16i = jnp.zeros((16,), jnp.int32)
    c15 = jnp.full((16,), 15, jnp.int32)

    def _zrow(i, carry):
        for jj in range(16):
            zb256[i, pl.ds(16 * jj, 16)] = jnp.zeros((16,), jnp.float32)
        return carry
    lax.fori_loop(0, 32, _zrow, 0)

    def _zrow16(i, carry):
        zb16[i, :] = jnp.zeros((16,), jnp.float32)
        return carry
    lax.fori_loop(0, STR, _zrow16, 0)

    for j in range(NCH):
        g = c * NCH + j
        lo = g * CH
        hi = lo + CH

        for k in range((STR + 31) // 32):
            st = pl.multiple_of(jnp.minimum(sid * STR + 32 * k, CH - 32), 8)
            pltpu.sync_copy(zb256, acc.at[pl.ds(st, 32)])
        pltpu.sync_copy(zb16, s16acc.at[pl.ds(pl.multiple_of(sid * STR, 8), STR)])
        plsc.subcore_barrier()

        def p2_block(b, carry):
            off = base + b * BLK
            offr = pl.multiple_of(off // 128, 8)
            pltpu.sync_copy(src_hbm.at[pl.ds(offr, BLK // 128)], srcv)
            pltpu.sync_copy(dst_hbm.at[pl.ds(offr, BLK // 128)], dstv)

            def grp(i, st2):
                nrows, pcnt = st2
                pcntv = jnp.full((16,), pcnt, jnp.int32)
                psrc = pendbuf[0, :]
                pdl = pendbuf[1, :]
                peid = pendbuf[2, :]
                r = i // 8
                col = (i % 8) * 16
                dstw = dstv[r, pl.ds(col, 16)]
                srcw = srcv[r, pl.ds(col, 16)]
                gelo = jnp.minimum(jnp.maximum(dstw - lo + 1, 0), 1)
                lthi = jnp.minimum(jnp.maximum(hi - dstw, 0), 1)
                mi = gelo * lthi
                pf = mi
                for sh in (1, 2, 4, 8):
                    smask = jnp.minimum(jnp.maximum(lane - (sh - 1), 0), 1)
                    pf = pf + pf.at[jnp.maximum(lane - sh, 0)].get(mode="promise_in_bounds") * smask
                pos = jnp.full((16,), -1, jnp.int32)
                for step in (8, 4, 2, 1):
                    cand = pos + step
                    pfc = pf.at[cand].get(mode="promise_in_bounds")
                    adv = jnp.minimum(jnp.maximum(lane - pfc + 1, 0), 1)
                    pos = pos + step * adv
                perm = jnp.minimum(pos + 1, 15)
                eidw = jnp.full((16,), off + 16 * i, jnp.int32) + lane
                csrc = srcw.at[perm].get(mode="promise_in_bounds")
                cdl = (dstw - lo).at[perm].get(mode="promise_in_bounds")
                ceid = eidw.at[perm].get(mode="promise_in_bounds")
                sh_idx = jnp.maximum(lane - pcntv, 0)
                ge = jnp.minimum(jnp.maximum(lane - pcntv + 1, 0), 1)
                mg_src = csrc.at[sh_idx].get(mode="promise_in_bounds") * ge + psrc * (one - ge)
                mg_dl = cdl.at[sh_idx].get(mode="promise_in_bounds") * ge + pdl * (one - ge)
                mg_eid = ceid.at[sh_idx].get(mode="promise_in_bounds") * ge + peid * (one - ge)
                r2 = nrows // 8
                c2 = 16 * (nrows % 8)
                selsrc[r2, pl.ds(c2, 16)] = mg_src
                seldl[r2, pl.ds(c2, 16)] = mg_dl
                seleid[r2, pl.ds(c2, 16)] = mg_eid
                cnt = pf[15]
                newt_s = pcnt + cnt
                fi_s = jnp.minimum(jnp.maximum(newt_s - 15, 0), 1)
                fi = jnp.full((16,), fi_s, jnp.int32)
                lf_idx = jnp.minimum(lane + 16 - pcntv, 15)
                lf_src = csrc.at[lf_idx].get(mode="promise_in_bounds")
                lf_dl = cdl.at[lf_idx].get(mode="promise_in_bounds")
                lf_eid = ceid.at[lf_idx].get(mode="promise_in_bounds")
                pendbuf[0, :] = lf_src * fi + mg_src * (one - fi)
                pendbuf[1, :] = lf_dl * fi + mg_dl * (one - fi)
                pendbuf[2, :] = lf_eid * fi + mg_eid * (one - fi)
                return (nrows + fi_s, newt_s - 16 * fi_s)

            pendbuf[0, :] = zero16i
            pendbuf[1, :] = zero16i
            pendbuf[2, :] = padeid
            nrows, pcnt = lax.fori_loop(0, BLK // 16, grp,
                                        (jnp.int32(0), jnp.int32(0)))

            pcntv = jnp.full((16,), pcnt, jnp.int32)
            vm = jnp.minimum(jnp.maximum(pcntv - lane, 0), 1)
            r2 = nrows // 8
            c2 = 16 * (nrows % 8)
            selsrc[r2, pl.ds(c2, 16)] = pendbuf[0, :] * vm
            seldl[r2, pl.ds(c2, 16)] = pendbuf[1, :] * vm
            seleid[r2, pl.ds(c2, 16)] = pendbuf[2, :] * vm + padeid * (one - vm)
            for kk in range(1, 8):
                idxp = jnp.minimum(nrows + kk, NR * 8 - 1)
                rp = idxp // 8
                cp = 16 * (idxp % 8)
                selsrc[rp, pl.ds(cp, 16)] = zero16i
                seldl[rp, pl.ds(cp, 16)] = zero16i
                seleid[rp, pl.ds(cp, 16)] = padeid
            total = nrows * 16 + pcnt
            nb = (total + BATCH - 1) // BATCH

            def batch(k, carry2):
                pltpu.sync_copy(h_hbm.at[selsrc.at[k]], hrow)
                pltpu.sync_copy(exs_hbm.at[seleid.at[k]], exrow)

                def sc_edge(i, carry3):
                    ev = exrow[i, :]
                    for hh in range(HEADS):
                        bv = jnp.full((16,), ev[hh], jnp.float32)
                        for vv in range(4):
                            sl = pl.ds(hh * HID + vv * 16, 16)
                            hrow[i, sl] = hrow[i, sl] * bv
                    return carry3
                lax.fori_loop(0, BATCH, sc_edge, 0)
                pltpu.sync_copy(hrow, acc.at[seldl.at[k]], add=True)
                pltpu.sync_copy(exrow, s16acc.at[seldl.at[k]], add=True)
                return carry2

            lax.fori_loop(0, nb, batch, 0)
            return carry
        lax.fori_loop(0, NBLK_MSG, p2_block, 0)
        plsc.subcore_barrier()

        stw = pl.multiple_of(sid * STR, 8)
        dtw = pl.multiple_of(lo + sid * STR, 8)
        pltpu.sync_copy(acc.at[pl.ds(stw, STR)], num_hbm.at[pl.ds(dtw, STR)])
        pltpu.sync_copy(s16acc.at[pl.ds(stw, STR)], s16_hbm.at[pl.ds(dtw, STR)])
        plsc.subcore_barrier()


def _sc_msg(src2d, dst2d, exs, h):
    mesh = plsc.VectorSubcoreMesh(core_axis_name="c", subcore_axis_name="s")
    fn = pl.kernel(
        _msg_body,
        out_type=[
            jax.ShapeDtypeStruct((NP, F4), jnp.float32),
            jax.ShapeDtypeStruct((NP, 16), jnp.float32),
        ],
        mesh=mesh,
        compiler_params=pltpu.CompilerParams(use_tc_tiling_on_sc=False),
        scratch_types=[
            pltpu.VMEM((BLK // 128, 128), jnp.int32),
            pltpu.VMEM((BLK // 128, 128), jnp.int32),
            pltpu.VMEM((NR, 128), jnp.int32),
            pltpu.VMEM((NR, 128), jnp.int32),
            pltpu.VMEM((NR, 128), jnp.int32),
            pltpu.VMEM((BATCH, F4), jnp.float32),
            pltpu.VMEM((BATCH, 16), jnp.float32),
            pltpu.VMEM((32, F4), jnp.float32),
            pltpu.VMEM((STR, 16), jnp.float32),
            pltpu.VMEM((4, 16), jnp.int32),
            pltpu.VMEM_SHARED((CH, F4), jnp.float32),
            pltpu.VMEM_SHARED((CH, 16), jnp.float32),
        ],
    )
    return fn(src2d, dst2d, exs, h)


def _finalize_body(num_ref, s16_ref, b_ref, out_ref):
    B = num_ref.shape[0]
    s = s16_ref[...][:, :HEADS]
    out = num_ref[...].reshape(B, HEADS, HID) / (s[:, :, None] + 1e-16)
    m = jnp.mean(out, axis=1) + b_ref[...][None, :]
    out_ref[...] = jnp.where(m > 0, m, jnp.exp(jnp.minimum(m, 0.0)) - 1.0)


def _finalize(num, s16, b):
    B = NP // 16
    return pl.pallas_call(
        _finalize_body,
        grid=(16,),
        in_specs=[
            pl.BlockSpec((B, F4), lambda i: (i, 0)),
            pl.BlockSpec((B, 16), lambda i: (i, 0)),
            pl.BlockSpec((HID,), lambda i: (0,)),
        ],
        out_specs=pl.BlockSpec((B, HID), lambda i: (i, 0)),
        out_shape=jax.ShapeDtypeStruct((NP, HID), jnp.float32),
    )(num, s16, b)


def _scorer_body(h_ref, demb_ref, s1w_ref, s1b_ref, s2w_ref, s2b_ref, out_ref):
    h = h_ref[...]
    ctx_w = s1w_ref[...]
    hid = jnp.dot(h, ctx_w[:HID], preferred_element_type=jnp.float32)
    hid += jnp.dot(demb_ref[...][None, :], ctx_w[HID:], preferred_element_type=jnp.float32)
    hid = jax.nn.relu(hid + s1b_ref[...][None, :])
    out_ref[...] = (jnp.dot(hid, s2w_ref[...], preferred_element_type=jnp.float32)
                    + s2b_ref[...][None, :])


def _scorer(h, donor_emb, S1w, S1b, S2w, S2b):
    B = NP // 16
    return pl.pallas_call(
        _scorer_body,
        grid=(16,),
        in_specs=[
            pl.BlockSpec((B, HID), lambda i: (i, 0)),
            pl.BlockSpec((CTX,), lambda i: (0,)),
            pl.BlockSpec((HID + CTX, HID), lambda i: (0, 0)),
            pl.BlockSpec((HID,), lambda i: (0,)),
            pl.BlockSpec((HID, 1), lambda i: (0, 0)),
            pl.BlockSpec((1,), lambda i: (0,)),
        ],
        out_specs=pl.BlockSpec((B, 1), lambda i: (i, 0)),
        out_shape=jax.ShapeDtypeStruct((NP, 1), jnp.float32),
    )(h, donor_emb, S1w, S1b, S2w, S2b)


def _gat_layer_sc(src2d, dst2d, xp, W, a_src, a_dst, b):
    h, als16, ald16, ms, md = _proj(xp, W, a_src, a_dst)
    exs = _sc_att(src2d, dst2d, als16, ald16, ms, md)
    num, s16 = _sc_msg(src2d, dst2d, exs, h)
    return _finalize(num, s16, b)


def kernel(x, edge_index, donor_emb, W1, a_src1, a_dst1, b1, W2, a_src2, a_dst2, b2, S1w, S1b, S2w, S2b):
    loops = jnp.arange(N, dtype=jnp.int32)
    pad = jnp.full((EP - EDGES,), N, jnp.int32)
    src2d = jnp.concatenate([edge_index[0].astype(jnp.int32), loops, pad]).reshape(EP // 128, 128)
    dst2d = jnp.concatenate([edge_index[1].astype(jnp.int32), loops, pad]).reshape(EP // 128, 128)

    xp = jnp.zeros((NP, NODE_DIM), jnp.float32).at[:N].set(x)
    hm1 = _gat_layer_sc(src2d, dst2d, xp, W1, a_src1, a_dst1, b1)
    hm2 = _gat_layer_sc(src2d, dst2d, hm1, W2, a_src2, a_dst2, b2)
    logits = _scorer(hm2, donor_emb, S1w, S1b, S2w, S2b)
    return logits[:N, 0]


# speedup vs baseline: 4.7250x; 1.0002x over previous
import jax
import jax.numpy as jnp
from jax import lax
from jax.experimental import pallas as pl
from jax.experimental.pallas import tpu as pltpu
from jax.experimental.pallas import tpu_sc as plsc

N = 50000
NP = 50176
E = 800000
EDGES = E + N
NODE_DIM = 27
HID = 64
CTX = 32
HEADS = 4
F4 = HEADS * HID

BLK = 2048
EP = 851968
NBLK_ATT = 13
NBLK_MSG = 26
SH_MSG = BLK * NBLK_MSG
CH = 3584
NCH = 7
STR = CH // 16
BATCH = 128
NR = 18
NEG = -1e30


def _proj_body(x_ref, w_ref, asrc_ref, adst_ref,
               h_ref, als_ref, ald_ref, maxs_ref, maxd_ref):
    i = pl.program_id(0)
    B = x_ref.shape[0]
    h = jnp.dot(x_ref[...], w_ref[...], preferred_element_type=jnp.float32)
    h_ref[...] = h
    hh = h.reshape(B, HEADS, HID)
    als = jnp.sum(hh * asrc_ref[...][None], axis=-1)
    ald = jnp.sum(hh * adst_ref[...][None], axis=-1)
    row = i * B + lax.broadcasted_iota(jnp.int32, (B, 1), 0)
    valid = row < N
    als16 = jnp.concatenate([als, jnp.zeros((B, 16 - HEADS), jnp.float32)], axis=1)
    ald16 = jnp.concatenate([ald, jnp.zeros((B, 16 - HEADS), jnp.float32)], axis=1)
    als16 = jnp.where(valid, als16, NEG)
    ald16 = jnp.where(valid, ald16, NEG)
    als_ref[...] = als16
    ald_ref[...] = ald16
    ms = jnp.max(als16, axis=0)
    md = jnp.max(ald16, axis=0)

    @pl.when(i == 0)
    def _():
        maxs_ref[...] = ms
        maxd_ref[...] = md

    @pl.when(i > 0)
    def _():
        maxs_ref[...] = jnp.maximum(maxs_ref[...], ms)
        maxd_ref[...] = jnp.maximum(maxd_ref[...], md)


def _proj(xp, W, a_src, a_dst):
    B = NP // 16
    D = xp.shape[1]
    return pl.pallas_call(
        _proj_body,
        grid=(16,),
        in_specs=[
            pl.BlockSpec((B, D), lambda i: (i, 0)),
            pl.BlockSpec((D, F4), lambda i: (0, 0)),
            pl.BlockSpec((HEADS, HID), lambda i: (0, 0)),
            pl.BlockSpec((HEADS, HID), lambda i: (0, 0)),
        ],
        out_specs=[
            pl.BlockSpec((B, F4), lambda i: (i, 0)),
            pl.BlockSpec((B, 16), lambda i: (i, 0)),
            pl.BlockSpec((B, 16), lambda i: (i, 0)),
            pl.BlockSpec((16,), lambda i: (0,)),
            pl.BlockSpec((16,), lambda i: (0,)),
        ],
        out_shape=[
            jax.ShapeDtypeStruct((NP, F4), jnp.float32),
            jax.ShapeDtypeStruct((NP, 16), jnp.float32),
            jax.ShapeDtypeStruct((NP, 16), jnp.float32),
            jax.ShapeDtypeStruct((16,), jnp.float32),
            jax.ShapeDtypeStruct((16,), jnp.float32),
        ],
    )(xp, W, a_src, a_dst)


def _att_body(src_hbm, dst_hbm, als_hbm, ald_hbm, maxs_hbm, maxd_hbm,
              exs_hbm,
              srcv, dstv, abuf, bbuf, exbuf, mbuf):
    c = lax.axis_index("c")
    sid = lax.axis_index("s")
    wid = sid * 2 + c
    base = wid * (EP // 32)

    pltpu.sync_copy(maxs_hbm, mbuf)
    ms = mbuf[...]
    pltpu.sync_copy(maxd_hbm, mbuf)
    md = mbuf[...]
    w = ms + md
    cub = jnp.where(w > 0, w, 0.2 * w)

    def p1_block(b, carry):
        offr = pl.multiple_of(base // 128 + b * (BLK // 128), 8)
        pltpu.sync_copy(src_hbm.at[pl.ds(offr, BLK // 128)], srcv)
        pltpu.sync_copy(dst_hbm.at[pl.ds(offr, BLK // 128)], dstv)
        for r in range(BLK // 128):
            pltpu.sync_copy(als_hbm.at[srcv.at[r]], abuf.at[pl.ds(128 * r, 128)])
            pltpu.sync_copy(ald_hbm.at[dstv.at[r]], bbuf.at[pl.ds(128 * r, 128)])

        def edge(i, carry2):
            v = abuf[i, :] + bbuf[i, :]
            v = jnp.where(v > 0, v, 0.2 * v) - cub
            exbuf[i, :] = jnp.exp(v)
            return carry2
        lax.fori_loop(0, BLK, edge, 0)
        pltpu.sync_copy(exbuf, exs_hbm.at[pl.ds(pl.multiple_of(base + b * BLK, 8), BLK)])
        return carry
    lax.fori_loop(0, NBLK_ATT, p1_block, 0)


def _sc_att(src2d, dst2d, als16, ald16, maxs, maxd):
    mesh = plsc.VectorSubcoreMesh(core_axis_name="c", subcore_axis_name="s")
    fn = pl.kernel(
        _att_body,
        out_type=[jax.ShapeDtypeStruct((EP, 16), jnp.float32)],
        mesh=mesh,
        compiler_params=pltpu.CompilerParams(use_tc_tiling_on_sc=False),
        scratch_types=[
            pltpu.VMEM((BLK // 128, 128), jnp.int32),
            pltpu.VMEM((BLK // 128, 128), jnp.int32),
            pltpu.VMEM((BLK, 16), jnp.float32),
            pltpu.VMEM((BLK, 16), jnp.float32),
            pltpu.VMEM((BLK, 16), jnp.float32),
            pltpu.VMEM((16,), jnp.float32),
        ],
    )
    return fn(src2d, dst2d, als16, ald16, maxs, maxd)[0]


def _msg_body(src_hbm, dst_hbm, exs_hbm, h_hbm,
              num_hbm, s16_hbm,
              srcv, dstv, selsrc, seldl, seleid, hrow, exrow, zb256, zb16,
              pendbuf, sem1, sem2, acc, s16acc):
    c = lax.axis_index("c")
    sid = lax.axis_index("s")
    base = sid * SH_MSG
    lane = lax.iota(jnp.int32, 16)
    one = jnp.ones((16,), jnp.int32)
    padeid = jnp.full((16,), EP - 1, jnp.int32)
    zero16i = jnp.zeros((16,), jnp.int32)
    c15 = jnp.full((16,), 15, jnp.int32)

    def _zrow(i, carry):
        for jj in range(16):
            zb256[i, pl.ds(16 * jj, 16)] = jnp.zeros((16,), jnp.float32)
        return carry
    lax.fori_loop(0, 32, _zrow, 0)

    def _zrow16(i, carry):
        zb16[i, :] = jnp.zeros((16,), jnp.float32)
        return carry
    lax.fori_loop(0, STR, _zrow16, 0)

    for j in range(NCH):
        g = c * NCH + j
        lo = g * CH
        hi = lo + CH

        for k in range((STR + 31) // 32):
            st = pl.multiple_of(jnp.minimum(sid * STR + 32 * k, CH - 32), 8)
            pltpu.sync_copy(zb256, acc.at[pl.ds(st, 32)])
        pltpu.sync_copy(zb16, s16acc.at[pl.ds(pl.multiple_of(sid * STR, 8), STR)])
        plsc.subcore_barrier()

        def p2_block(b, carry):
            off = base + b * BLK
            offr = pl.multiple_of(off // 128, 8)
            pltpu.sync_copy(src_hbm.at[pl.ds(offr, BLK // 128)], srcv)
            pltpu.sync_copy(dst_hbm.at[pl.ds(offr, BLK // 128)], dstv)

            def grp(i, st2):
                nrows, pcnt = st2
                pcntv = jnp.full((16,), pcnt, jnp.int32)
                psrc = pendbuf[0, :]
                pdl = pendbuf[1, :]
                peid = pendbuf[2, :]
                r = i // 8
                col = (i % 8) * 16
                dstw = dstv[r, pl.ds(col, 16)]
                srcw = srcv[r, pl.ds(col, 16)]
                gelo = jnp.minimum(jnp.maximum(dstw - lo + 1, 0), 1)
                lthi = jnp.minimum(jnp.maximum(hi - dstw, 0), 1)
                mi = gelo * lthi
                pf = mi
                for sh in (1, 2, 4, 8):
                    smask = jnp.minimum(jnp.maximum(lane - (sh - 1), 0), 1)
                    pf = pf + pf.at[jnp.maximum(lane - sh, 0)].get(mode="promise_in_bounds") * smask
                pos = jnp.full((16,), -1, jnp.int32)
                for step in (8, 4, 2, 1):
                    cand = pos + step
                    pfc = pf.at[cand].get(mode="promise_in_bounds")
                    adv = jnp.minimum(jnp.maximum(lane - pfc + 1, 0), 1)
                    pos = pos + step * adv
                perm = jnp.minimum(pos + 1, 15)
                eidw = jnp.full((16,), off + 16 * i, jnp.int32) + lane
                csrc = srcw.at[perm].get(mode="promise_in_bounds")
                cdl = (dstw - lo).at[perm].get(mode="promise_in_bounds")
                ceid = eidw.at[perm].get(mode="promise_in_bounds")
                sh_idx = jnp.maximum(lane - pcntv, 0)
                ge = jnp.minimum(jnp.maximum(lane - pcntv + 1, 0), 1)
                mg_src = csrc.at[sh_idx].get(mode="promise_in_bounds") * ge + psrc * (one - ge)
                mg_dl = cdl.at[sh_idx].get(mode="promise_in_bounds") * ge + pdl * (one - ge)
                mg_eid = ceid.at[sh_idx].get(mode="promise_in_bounds") * ge + peid * (one - ge)
                r2 = nrows // 8
                c2 = 16 * (nrows % 8)
                selsrc[r2, pl.ds(c2, 16)] = mg_src
                seldl[r2, pl.ds(c2, 16)] = mg_dl
                seleid[r2, pl.ds(c2, 16)] = mg_eid
                cnt = pf[15]
                newt_s = pcnt + cnt
                fi_s = jnp.minimum(jnp.maximum(newt_s - 15, 0), 1)
                fi = jnp.full((16,), fi_s, jnp.int32)
                lf_idx = jnp.minimum(lane + 16 - pcntv, 15)
                lf_src = csrc.at[lf_idx].get(mode="promise_in_bounds")
                lf_dl = cdl.at[lf_idx].get(mode="promise_in_bounds")
                lf_eid = ceid.at[lf_idx].get(mode="promise_in_bounds")
                pendbuf[0, :] = lf_src * fi + mg_src * (one - fi)
                pendbuf[1, :] = lf_dl * fi + mg_dl * (one - fi)
                pendbuf[2, :] = lf_eid * fi + mg_eid * (one - fi)
                return (nrows + fi_s, newt_s - 16 * fi_s)

            pendbuf[0, :] = zero16i
            pendbuf[1, :] = zero16i
            pendbuf[2, :] = padeid
            nrows, pcnt = lax.fori_loop(0, BLK // 16, grp,
                                        (jnp.int32(0), jnp.int32(0)))

            pcntv = jnp.full((16,), pcnt, jnp.int32)
            vm = jnp.minimum(jnp.maximum(pcntv - lane, 0), 1)
            r2 = nrows // 8
            c2 = 16 * (nrows % 8)
            selsrc[r2, pl.ds(c2, 16)] = pendbuf[0, :] * vm
            seldl[r2, pl.ds(c2, 16)] = pendbuf[1, :] * vm
            seleid[r2, pl.ds(c2, 16)] = pendbuf[2, :] * vm + padeid * (one - vm)
            for kk in range(1, 8):
                idxp = jnp.minimum(nrows + kk, NR * 8 - 1)
                rp = idxp // 8
                cp = 16 * (idxp % 8)
                selsrc[rp, pl.ds(cp, 16)] = zero16i
                seldl[rp, pl.ds(cp, 16)] = zero16i
                seleid[rp, pl.ds(cp, 16)] = padeid
            total = nrows * 16 + pcnt
            nb = (total + BATCH - 1) // BATCH

            def batch(k, carry2):
                d1 = pltpu.async_copy(h_hbm.at[selsrc.at[k]], hrow, sem1)
                d2 = pltpu.async_copy(exs_hbm.at[seleid.at[k]], exrow, sem2)
                d1.wait()
                d2.wait()

                def sc_edge(i, carry3):
                    ev = exrow[i, :]
                    for hh in range(HEADS):
                        bv = jnp.full((16,), ev[hh], jnp.float32)
                        for vv in range(4):
                            sl = pl.ds(hh * HID + vv * 16, 16)
                            hrow[i, sl] = hrow[i, sl] * bv
                    return carry3
                lax.fori_loop(0, BATCH, sc_edge, 0)
                d3 = pltpu.async_copy(hrow, acc.at[seldl.at[k]], sem1, add=True)
                d4 = pltpu.async_copy(exrow, s16acc.at[seldl.at[k]], sem2, add=True)
                d3.wait()
                d4.wait()
                return carry2

            lax.fori_loop(0, nb, batch, 0)
            return carry
        lax.fori_loop(0, NBLK_MSG, p2_block, 0)
        plsc.subcore_barrier()

        stw = pl.multiple_of(sid * STR, 8)
        dtw = pl.multiple_of(lo + sid * STR, 8)
        pltpu.sync_copy(acc.at[pl.ds(stw, STR)], num_hbm.at[pl.ds(dtw, STR)])
        pltpu.sync_copy(s16acc.at[pl.ds(stw, STR)], s16_hbm.at[pl.ds(dtw, STR)])
        plsc.subcore_barrier()


def _sc_msg(src2d, dst2d, exs, h):
    mesh = plsc.VectorSubcoreMesh(core_axis_name="c", subcore_axis_name="s")
    fn = pl.kernel(
        _msg_body,
        out_type=[
            jax.ShapeDtypeStruct((NP, F4), jnp.float32),
            jax.ShapeDtypeStruct((NP, 16), jnp.float32),
        ],
        mesh=mesh,
        compiler_params=pltpu.CompilerParams(use_tc_tiling_on_sc=False),
        scratch_types=[
            pltpu.VMEM((BLK // 128, 128), jnp.int32),
            pltpu.VMEM((BLK // 128, 128), jnp.int32),
            pltpu.VMEM((NR, 128), jnp.int32),
            pltpu.VMEM((NR, 128), jnp.int32),
            pltpu.VMEM((NR, 128), jnp.int32),
            pltpu.VMEM((BATCH, F4), jnp.float32),
            pltpu.VMEM((BATCH, 16), jnp.float32),
            pltpu.VMEM((32, F4), jnp.float32),
            pltpu.VMEM((STR, 16), jnp.float32),
            pltpu.VMEM((4, 16), jnp.int32),
            pltpu.SemaphoreType.DMA,
            pltpu.SemaphoreType.DMA,
            pltpu.VMEM_SHARED((CH, F4), jnp.float32),
            pltpu.VMEM_SHARED((CH, 16), jnp.float32),
        ],
    )
    return fn(src2d, dst2d, exs, h)


def _finalize_body(num_ref, s16_ref, b_ref, out_ref):
    B = num_ref.shape[0]
    s = s16_ref[...][:, :HEADS]
    out = num_ref[...].reshape(B, HEADS, HID) / (s[:, :, None] + 1e-16)
    m = jnp.mean(out, axis=1) + b_ref[...][None, :]
    out_ref[...] = jnp.where(m > 0, m, jnp.exp(jnp.minimum(m, 0.0)) - 1.0)


def _finalize(num, s16, b):
    B = NP // 16
    return pl.pallas_call(
        _finalize_body,
        grid=(16,),
        in_specs=[
            pl.BlockSpec((B, F4), lambda i: (i, 0)),
            pl.BlockSpec((B, 16), lambda i: (i, 0)),
            pl.BlockSpec((HID,), lambda i: (0,)),
        ],
        out_specs=pl.BlockSpec((B, HID), lambda i: (i, 0)),
        out_shape=jax.ShapeDtypeStruct((NP, HID), jnp.float32),
    )(num, s16, b)


def _scorer_body(h_ref, demb_ref, s1w_ref, s1b_ref, s2w_ref, s2b_ref, out_ref):
    h = h_ref[...]
    ctx_w = s1w_ref[...]
    hid = jnp.dot(h, ctx_w[:HID], preferred_element_type=jnp.float32)
    hid += jnp.dot(demb_ref[...][None, :], ctx_w[HID:], preferred_element_type=jnp.float32)
    hid = jax.nn.relu(hid + s1b_ref[...][None, :])
    out_ref[...] = (jnp.dot(hid, s2w_ref[...], preferred_element_type=jnp.float32)
                    + s2b_ref[...][None, :])


def _scorer(h, donor_emb, S1w, S1b, S2w, S2b):
    B = NP // 16
    return pl.pallas_call(
        _scorer_body,
        grid=(16,),
        in_specs=[
            pl.BlockSpec((B, HID), lambda i: (i, 0)),
            pl.BlockSpec((CTX,), lambda i: (0,)),
            pl.BlockSpec((HID + CTX, HID), lambda i: (0, 0)),
            pl.BlockSpec((HID,), lambda i: (0,)),
            pl.BlockSpec((HID, 1), lambda i: (0, 0)),
            pl.BlockSpec((1,), lambda i: (0,)),
        ],
        out_specs=pl.BlockSpec((B, 1), lambda i: (i, 0)),
        out_shape=jax.ShapeDtypeStruct((NP, 1), jnp.float32),
    )(h, donor_emb, S1w, S1b, S2w, S2b)


def _gat_layer_sc(src2d, dst2d, xp, W, a_src, a_dst, b):
    h, als16, ald16, ms, md = _proj(xp, W, a_src, a_dst)
    exs = _sc_att(src2d, dst2d, als16, ald16, ms, md)
    num, s16 = _sc_msg(src2d, dst2d, exs, h)
    return _finalize(num, s16, b)


def kernel(x, edge_index, donor_emb, W1, a_src1, a_dst1, b1, W2, a_src2, a_dst2, b2, S1w, S1b, S2w, S2b):
    loops = jnp.arange(N, dtype=jnp.int32)
    pad = jnp.full((EP - EDGES,), N, jnp.int32)
    src2d = jnp.concatenate([edge_index[0].astype(jnp.int32), loops, pad]).reshape(EP // 128, 128)
    dst2d = jnp.concatenate([edge_index[1].astype(jnp.int32), loops, pad]).reshape(EP // 128, 128)

    xp = jnp.zeros((NP, NODE_DIM), jnp.float32).at[:N].set(x)
    hm1 = _gat_layer_sc(src2d, dst2d, xp, W1, a_src1, a_dst1, b1)
    hm2 = _gat_layer_sc(src2d, dst2d, hm1, W2, a_src2, a_dst2, b2)
    logits = _scorer(hm2, donor_emb, S1w, S1b, S2w, S2b)
    return logits[:N, 0]
